# baseline (speedup 1.0000x reference)
import functools

import jax
import jax.numpy as jnp
from jax import lax
from jax.experimental import pallas as pl
from jax.experimental.pallas import tpu as pltpu
from jax.experimental.pallas import tpu_sc as plsc

_V = 1000
_NTOK = 16 * 2048
_NC = 2
_NS = 16
_L = 16
_NW = _NC * _NS
_BPW = _NTOK // _NW
_CH = 32
_NCHUNK = _BPW // _CH
_NBUF = 3
_VP = 1024


def _lse_body(table_ref, out_ref):
    x = table_ref[...]
    m = jnp.max(x, axis=1, keepdims=True)
    s = jnp.sum(jnp.exp(x - m), axis=1, keepdims=True)
    out_ref[...] = m + jnp.log(s)


def _row_lse(table):
    return pl.pallas_call(
        _lse_body,
        out_shape=jax.ShapeDtypeStruct((_V, 1), jnp.float32),
    )(table)


_LB = 128


def _lseg_body(idx_ref, lse_ref, out_ref):
    onehot = (idx_ref[...] ==
              lax.broadcasted_iota(jnp.int32, (_LB, _V), 1)
              ).astype(jnp.float32)
    out_ref[...] = lax.dot_general(
        onehot, lse_ref[...], (((1,), (0,)), ((), ())),
        preferred_element_type=jnp.float32)


def _lse_gather(idx1, lse1):
    return pl.pallas_call(
        _lseg_body,
        grid=(_NTOK // _LB,),
        in_specs=[pl.BlockSpec((_LB, 1), lambda i: (i, 0)),
                  pl.BlockSpec((_V, 1), lambda i: (0, 0))],
        out_specs=pl.BlockSpec((_LB, 1), lambda i: (i, 0)),
        out_shape=jax.ShapeDtypeStruct((_NTOK, 1), jnp.float32),
    )(idx1, lse1)


@functools.partial(
    pl.kernel,
    out_type=(
        jax.ShapeDtypeStruct((_NTOK, _VP), jnp.float32),
        jax.ShapeDtypeStruct((_NW * _L,), jnp.float32),
    ),
    mesh=plsc.VectorSubcoreMesh(core_axis_name="c", subcore_axis_name="s"),
    compiler_params=pltpu.CompilerParams(use_tc_tiling_on_sc=True),
    scratch_types=(
        pltpu.VMEM((_BPW,), jnp.int32),
        pltpu.VMEM((_BPW,), jnp.int32),
        pltpu.VMEM((_BPW,), jnp.float32),
        pltpu.VMEM((_CH, _VP), jnp.float32),
        pltpu.VMEM((_CH, _VP), jnp.float32),
        pltpu.VMEM((_CH, _VP), jnp.float32),
        pltpu.SemaphoreType.DMA,
        pltpu.SemaphoreType.DMA,
        pltpu.SemaphoreType.DMA,
        pltpu.SemaphoreType.DMA,
        pltpu.SemaphoreType.DMA,
        pltpu.SemaphoreType.DMA,
        pltpu.SemaphoreType.DMA,
    ),
)
def _sc_lookup(table, tflat, idxf, tgtf, out, part,
               idx_v, lin_v, tval_v, rows_0, rows_1, rows_2,
               gsem_0, gsem_1, gsem_2, ssem_0, ssem_1, ssem_2, lsem):
    c = lax.axis_index("c")
    s = lax.axis_index("s")
    wid = s * _NC + c
    base = wid * _BPW

    rows = (rows_0, rows_1, rows_2)
    gsem = (gsem_0, gsem_1, gsem_2)
    ssem = (ssem_0, ssem_1, ssem_2)

    def gather(n):
        b = n % _NBUF
        return pltpu.make_async_copy(
            table.at[idx_v.at[pl.ds(n * _CH, _CH)]], rows[b], gsem[b])

    def scatter(n):
        b = n % _NBUF
        return pltpu.make_async_copy(
            rows[b], out.at[pl.ds(base + n * _CH, _CH)], ssem[b])

    pltpu.sync_copy(idxf.at[pl.ds(base, _BPW)], idx_v)
    pltpu.sync_copy(tgtf.at[pl.ds(base, _BPW)], lin_v)
    gather(0).start()
    gather(1).start()

    for n in range(_NCHUNK):
        if n + 2 < _NCHUNK:
            if n >= 1:
                scatter(n - 1).wait()
            gather(n + 2).start()
        gather(n).wait()
        scatter(n).start()

    for j in range(_BPW // _L):
        sl = pl.ds(j * _L, _L)
        lin_v[sl] = idx_v[sl] * _V + lin_v[sl]
    pltpu.async_copy(tflat.at[lin_v], tval_v, lsem).wait()
    acc = jnp.zeros((_L,), jnp.float32)
    for j in range(_BPW // _L):
        sl = pl.ds(j * _L, _L)
        acc = acc + tval_v[sl]
    tval_v[pl.ds(0, _L)] = acc
    pltpu.sync_copy(tval_v.at[pl.ds(0, _L)], part.at[pl.ds(wid * _L, _L)])

    scatter(_NCHUNK - 2).wait()
    scatter(_NCHUNK - 1).wait()


def kernel(idx, targets, table):
    idxf = idx.reshape(-1).astype(jnp.int32)
    tgtf = targets.reshape(-1).astype(jnp.int32)
    table = table.astype(jnp.float32)
    lse1 = _row_lse(table)
    tflat = jnp.pad(table.reshape(-1), (0, 8))
    table_p = jnp.pad(table, ((0, 0), (0, _VP - _V)))
    gathered, part = _sc_lookup(table_p, tflat, idxf, tgtf)
    lse_tok = _lse_gather(idxf.reshape(_NTOK, 1), lse1)
    loss = (jnp.sum(lse_tok) - jnp.sum(part)) / _NTOK
    return (gathered[:, :_V], loss)


# speedup vs baseline: 1.4563x; 1.4563x over previous
import functools

import jax
import jax.numpy as jnp
from jax import lax
from jax.experimental import pallas as pl
from jax.experimental.pallas import tpu as pltpu
from jax.experimental.pallas import tpu_sc as plsc

_V = 1000
_NTOK = 16 * 2048
_NC = 2
_NS = 16
_L = 16
_NW = _NC * _NS
_BPW = _NTOK // _NW
_CH = 32
_NCHUNK = _BPW // _CH
_NBUF = 3
_VP = 1024


def _lse_body(table_ref, out_ref):
    x = table_ref[...]
    m = jnp.max(x, axis=1, keepdims=True)
    s = jnp.sum(jnp.exp(x - m), axis=1, keepdims=True)
    out_ref[...] = m + jnp.log(s)


def _row_lse(table):
    return pl.pallas_call(
        _lse_body,
        out_shape=jax.ShapeDtypeStruct((_V, 1), jnp.float32),
    )(table)


_TB = 256


def _tr_body(in_ref, out_ref):
    out_ref[...] = in_ref[:, : _V].T


def _transpose(x):
    return pl.pallas_call(
        _tr_body,
        grid=(_NTOK // _TB,),
        in_specs=[pl.BlockSpec((_TB, _VP), lambda i: (i, 0))],
        out_specs=pl.BlockSpec((_V, _TB), lambda i: (0, i)),
        out_shape=jax.ShapeDtypeStruct((_V, _NTOK), jnp.float32),
    )(x)


@functools.partial(
    pl.kernel,
    out_type=(
        jax.ShapeDtypeStruct((_NTOK, _VP), jnp.float32),
        jax.ShapeDtypeStruct((_NW * _L,), jnp.float32),
    ),
    mesh=plsc.VectorSubcoreMesh(core_axis_name="c", subcore_axis_name="s"),
    compiler_params=pltpu.CompilerParams(use_tc_tiling_on_sc=True),
    scratch_types=(
        pltpu.VMEM((_BPW,), jnp.int32),
        pltpu.VMEM((_BPW,), jnp.int32),
        pltpu.VMEM((_BPW,), jnp.float32),
        pltpu.VMEM((_BPW,), jnp.float32),
        pltpu.VMEM((_CH, _VP), jnp.float32),
        pltpu.VMEM((_CH, _VP), jnp.float32),
        pltpu.VMEM((_CH, _VP), jnp.float32),
        pltpu.SemaphoreType.DMA,
        pltpu.SemaphoreType.DMA,
        pltpu.SemaphoreType.DMA,
        pltpu.SemaphoreType.DMA,
        pltpu.SemaphoreType.DMA,
        pltpu.SemaphoreType.DMA,
        pltpu.SemaphoreType.DMA,
    ),
)
def _sc_lookup(table, tflat, idxf, tgtf, lse, out, part,
               idx_v, lin_v, tval_v, lval_v, rows_0, rows_1, rows_2,
               gsem_0, gsem_1, gsem_2, ssem_0, ssem_1, ssem_2, lsem):
    c = lax.axis_index("c")
    s = lax.axis_index("s")
    wid = s * _NC + c
    base = wid * _BPW

    rows = (rows_0, rows_1, rows_2)
    gsem = (gsem_0, gsem_1, gsem_2)
    ssem = (ssem_0, ssem_1, ssem_2)

    def gather(n):
        b = n % _NBUF
        return pltpu.make_async_copy(
            table.at[idx_v.at[pl.ds(n * _CH, _CH)]], rows[b], gsem[b])

    def scatter(n):
        b = n % _NBUF
        return pltpu.make_async_copy(
            rows[b], out.at[pl.ds(base + n * _CH, _CH)], ssem[b])

    pltpu.sync_copy(idxf.at[pl.ds(base, _BPW)], idx_v)
    pltpu.sync_copy(tgtf.at[pl.ds(base, _BPW)], lin_v)
    gather(0).start()
    gather(1).start()

    for n in range(_NCHUNK):
        if n + 2 < _NCHUNK:
            if n >= 1:
                scatter(n - 1).wait()
            gather(n + 2).start()
        gather(n).wait()
        scatter(n).start()

    for j in range(_BPW // _L):
        sl = pl.ds(j * _L, _L)
        lin_v[sl] = idx_v[sl] * _V + lin_v[sl]
    pltpu.async_copy(tflat.at[lin_v], tval_v, lsem).wait()
    pltpu.async_copy(lse.at[idx_v], lval_v, lsem).wait()
    acc = jnp.zeros((_L,), jnp.float32)
    for j in range(_BPW // _L):
        sl = pl.ds(j * _L, _L)
        acc = acc + (lval_v[sl] - tval_v[sl])
    tval_v[pl.ds(0, _L)] = acc
    pltpu.sync_copy(tval_v.at[pl.ds(0, _L)], part.at[pl.ds(wid * _L, _L)])

    scatter(_NCHUNK - 2).wait()
    scatter(_NCHUNK - 1).wait()


def kernel(idx, targets, table):
    idxf = idx.reshape(-1).astype(jnp.int32)
    tgtf = targets.reshape(-1).astype(jnp.int32)
    table = table.astype(jnp.float32)
    lse = _row_lse(table).reshape(_V)
    tflat = jnp.pad(table.reshape(-1), (0, 8))
    table_p = jnp.pad(table, ((0, 0), (0, _VP - _V)))
    gathered, part = _sc_lookup(table_p, tflat, idxf, tgtf, lse)
    loss = jnp.sum(part) / _NTOK
    return (gathered[:, :_V], loss)


# speedup vs baseline: 1.4564x; 1.0001x over previous
import functools

import jax
import jax.numpy as jnp
from jax import lax
from jax.experimental import pallas as pl
from jax.experimental.pallas import tpu as pltpu
from jax.experimental.pallas import tpu_sc as plsc

_V = 1000
_NTOK = 16 * 2048
_NC = 2
_NS = 16
_L = 16
_NW = _NC * _NS
_BPW = _NTOK // _NW
_CH = 32
_NCHUNK = _BPW // _CH
_NBUF = 3
_VP = 1024


def _lse_body(table_ref, out_ref):
    x = table_ref[...]
    m = jnp.max(x, axis=1, keepdims=True)
    s = jnp.sum(jnp.exp(x - m), axis=1, keepdims=True)
    out_ref[...] = m + jnp.log(s)


def _row_lse(table):
    return pl.pallas_call(
        _lse_body,
        out_shape=jax.ShapeDtypeStruct((_V, 1), jnp.float32),
    )(table)


@functools.partial(
    pl.kernel,
    out_type=(
        jax.ShapeDtypeStruct((_NTOK, _VP), jnp.float32),
        jax.ShapeDtypeStruct((_NW * _L,), jnp.float32),
    ),
    mesh=plsc.VectorSubcoreMesh(core_axis_name="c", subcore_axis_name="s"),
    compiler_params=pltpu.CompilerParams(use_tc_tiling_on_sc=True),
    scratch_types=(
        pltpu.VMEM((_BPW,), jnp.int32),
        pltpu.VMEM((_BPW,), jnp.int32),
        pltpu.VMEM((_BPW,), jnp.float32),
        pltpu.VMEM((_BPW,), jnp.float32),
        pltpu.VMEM((_CH, _VP), jnp.float32),
        pltpu.VMEM((_CH, _VP), jnp.float32),
        pltpu.VMEM((_CH, _VP), jnp.float32),
        pltpu.SemaphoreType.DMA,
        pltpu.SemaphoreType.DMA,
        pltpu.SemaphoreType.DMA,
        pltpu.SemaphoreType.DMA,
        pltpu.SemaphoreType.DMA,
        pltpu.SemaphoreType.DMA,
        pltpu.SemaphoreType.DMA,
    ),
)
def _sc_lookup(table, tflat, idxf, tgtf, lse, out, part,
               idx_v, lin_v, tval_v, lval_v, rows_0, rows_1, rows_2,
               gsem_0, gsem_1, gsem_2, ssem_0, ssem_1, ssem_2, lsem):
    c = lax.axis_index("c")
    s = lax.axis_index("s")
    wid = s * _NC + c
    base = wid * _BPW

    rows = (rows_0, rows_1, rows_2)
    gsem = (gsem_0, gsem_1, gsem_2)
    ssem = (ssem_0, ssem_1, ssem_2)

    def gather(n):
        b = n % _NBUF
        return pltpu.make_async_copy(
            table.at[idx_v.at[pl.ds(n * _CH, _CH)]], rows[b], gsem[b])

    def scatter(n):
        b = n % _NBUF
        return pltpu.make_async_copy(
            rows[b], out.at[pl.ds(base + n * _CH, _CH)], ssem[b])

    pltpu.sync_copy(idxf.at[pl.ds(base, _BPW)], idx_v)
    pltpu.sync_copy(tgtf.at[pl.ds(base, _BPW)], lin_v)
    gather(0).start()
    gather(1).start()

    for n in range(_NCHUNK):
        if n + 2 < _NCHUNK:
            if n >= 1:
                scatter(n - 1).wait()
            gather(n + 2).start()
        gather(n).wait()
        scatter(n).start()

    for j in range(_BPW // _L):
        sl = pl.ds(j * _L, _L)
        lin_v[sl] = idx_v[sl] * _V + lin_v[sl]
    pltpu.async_copy(tflat.at[lin_v], tval_v, lsem).wait()
    pltpu.async_copy(lse.at[idx_v], lval_v, lsem).wait()
    acc = jnp.zeros((_L,), jnp.float32)
    for j in range(_BPW // _L):
        sl = pl.ds(j * _L, _L)
        acc = acc + (lval_v[sl] - tval_v[sl])
    tval_v[pl.ds(0, _L)] = acc
    pltpu.sync_copy(tval_v.at[pl.ds(0, _L)], part.at[pl.ds(wid * _L, _L)])

    scatter(_NCHUNK - 2).wait()
    scatter(_NCHUNK - 1).wait()


def kernel(idx, targets, table):
    idxf = idx.reshape(-1).astype(jnp.int32)
    tgtf = targets.reshape(-1).astype(jnp.int32)
    table = table.astype(jnp.float32)
    lse = _row_lse(table).reshape(_V)
    tflat = jnp.pad(table.reshape(-1), (0, 8))
    table_p = jnp.pad(table, ((0, 0), (0, _VP - _V)))
    gathered, part = _sc_lookup(table_p, tflat, idxf, tgtf, lse)
    loss = jnp.sum(part) / _NTOK
    return (gathered[:, :_V], loss)
